# Initial kernel scaffold; baseline (speedup 1.0000x reference)
import jax
import jax.numpy as jnp
from jax.experimental import pallas as pl


def kernel(segments):
    raise NotImplementedError("write your pallas kernel here")


# speedup vs baseline: 37.6919x; 37.6919x over previous
import functools

import jax
import jax.numpy as jnp
from jax import lax
from jax.experimental import pallas as pl
from jax.experimental.pallas import tpu as pltpu
from jax.experimental.pallas import tpu_sc as plsc

B = 8
H = 384
W = 384
N = 1024
NC = 2
NS = 16
ROWS_PER_TILE = H // NS
SEG_PER_TILE = ROWS_PER_TILE * W
PAIR_VECS = SEG_PER_TILE // 16
UNROLL = 8
J_ITERS = PAIR_VECS // UNROLL
NUM_IDX = 2 * PAIR_VECS * 16
ACC_PAD = 64
ACC = N * N + ACC_PAD
IMGS_PER_CORE = B // NC
SLICE = N * N // NS
ZCHUNK = 16384


def _adj_body(seg_hbm, out_hbm, segbuf, idx_buf, ones_buf, zbuf, acc):
    core = lax.axis_index("c")
    sid = lax.axis_index("s")
    iota = lax.iota(jnp.int32, 16)
    ones_v = jnp.full((16,), 1.0, dtype=jnp.float32)
    zero_v = jnp.zeros((16,), dtype=jnp.float32)

    def fill_ones(j, carry):
        ones_buf[pl.ds(j * 16, 16)] = ones_v
        return carry

    lax.fori_loop(0, NUM_IDX // 16, fill_ones, 0)

    def fill_zero(i, carry):
        zbuf[pl.ds(i * 16, 16)] = zero_v
        return carry

    lax.fori_loop(0, ZCHUNK // 16, fill_zero, 0)

    my_slice = pl.multiple_of(sid * SLICE, 8)

    for i in range(IMGS_PER_CORE):
        img = core * IMGS_PER_CORE + i

        seg_start = pl.multiple_of(sid * SEG_PER_TILE, 8)
        pltpu.sync_copy(
            seg_hbm.at[img, pl.ds(seg_start, SEG_PER_TILE)],
            segbuf.at[pl.ds(0, SEG_PER_TILE)],
        )

        for k in range(SLICE // ZCHUNK):
            pltpu.sync_copy(zbuf, acc.at[pl.ds(my_slice + k * ZCHUNK, ZCHUNK)])

        @pl.when(sid == 0)
        def _zero_pad():
            pltpu.sync_copy(zbuf.at[pl.ds(0, ACC_PAD)], acc.at[pl.ds(N * N, ACC_PAD)])

        def pair_chunks(j, carry):
            for u in range(UNROLL):
                p = j * (UNROLL * 16) + u * 16
                left = segbuf[pl.ds(pl.multiple_of(p, 16), 16)]
                right = segbuf[pl.ds(p + 1, 16)]
                pos = p + iota
                valid = (left != right) & (lax.rem(pos, W) != (W - 1))
                trash = N * N + iota
                fwd = jnp.where(valid, left * N + right, trash)
                bwd = jnp.where(valid, right * N + left, trash)
                idx_buf[pl.ds(p, 16)] = fwd
                idx_buf[pl.ds(SEG_PER_TILE + p, 16)] = bwd
            return carry

        lax.fori_loop(0, J_ITERS, pair_chunks, 0)

        plsc.subcore_barrier()

        pltpu.sync_copy(ones_buf, acc.at[idx_buf], add=True)

        plsc.subcore_barrier()

        pltpu.sync_copy(
            acc.at[pl.ds(my_slice, SLICE)],
            out_hbm.at[img, pl.ds(my_slice, SLICE)],
        )


@jax.jit
def _adjacency(seg_flat):
    mesh = plsc.VectorSubcoreMesh(
        core_axis_name="c", subcore_axis_name="s", num_cores=NC, num_subcores=NS
    )
    run = functools.partial(
        pl.kernel,
        out_type=jax.ShapeDtypeStruct((B, N * N), jnp.float32),
        mesh=mesh,
        scratch_types=[
            pltpu.VMEM((SEG_PER_TILE + 8,), jnp.int32),
            pltpu.VMEM((NUM_IDX,), jnp.int32),
            pltpu.VMEM((NUM_IDX,), jnp.float32),
            pltpu.VMEM((ZCHUNK,), jnp.float32),
            pltpu.VMEM_SHARED((ACC,), jnp.float32),
        ],
    )(_adj_body)
    return run(seg_flat)


def kernel(segments):
    if segments.ndim == 4:
        segments = segments[:, 0]
    seg_flat = segments.reshape(B, H * W).astype(jnp.int32)
    out = _adjacency(seg_flat)
    return out.reshape(B, N, N)
